# baseline (speedup 1.0000x reference)
import functools

import jax
import jax.numpy as jnp
from jax import lax
from jax.experimental import pallas as pl
from jax.experimental.pallas import tpu as pltpu
from jax.experimental.pallas import tpu_sc as plsc

_B = 16384
_D = 128
_NT = 12
_NC = 2
_NS = 16
_NW = _NC * _NS
_BPW = _B // _NW
_L = 16
_G = _BPW // _L
_WCHUNK = 128
_NWCH = _BPW // _WCHUNK
_GPC = _WCHUNK // _L


@functools.partial(
    pl.kernel,
    out_type=jax.ShapeDtypeStruct((_B * _D,), jnp.float32),
    mesh=plsc.VectorSubcoreMesh(core_axis_name="c", subcore_axis_name="s"),
    compiler_params=pltpu.CompilerParams(needs_layout_passes=False),
    scratch_types=[
        pltpu.VMEM((_BPW,), jnp.int32),
        pltpu.VMEM((_BPW,), jnp.int32),
        pltpu.VMEM((_BPW,), jnp.int32),
        pltpu.VMEM((32,), jnp.int32),
        pltpu.VMEM((_NT * _D,), jnp.float32),
        pltpu.VMEM((_BPW * _D,), jnp.float32),
        pltpu.SemaphoreType.DMA,
    ],
)
def _task_encode_sc(tm_hbm, pn_hbm, table_hbm, lut_hbm, out_hbm,
                    tm_v, pn_v, idx_v, lut_v, tab_v, rows_v, sem):
    wid = lax.axis_index("s") * _NC + lax.axis_index("c")
    base = wid * _BPW

    pltpu.sync_copy(tm_hbm.at[pl.ds(base, _BPW)], tm_v)
    pltpu.sync_copy(pn_hbm.at[pl.ds(base, _BPW)], pn_v)
    pltpu.sync_copy(lut_hbm, lut_v)
    pltpu.sync_copy(table_hbm, tab_v)

    def lut_body(i, carry):
        tm = tm_v[pl.ds(i * _L, _L)]
        pn = pn_v[pl.ds(i * _L, _L)]
        idx_v[pl.ds(i * _L, _L)] = plsc.load_gather(lut_v, [tm * 3 + pn])
        return carry

    lax.fori_loop(0, _G, lut_body, jnp.int32(0), unroll=4)

    copies = []
    for ch in range(_NWCH):
        def grp(g, carry, ch=ch):
            gg = ch * _GPC + g
            task = idx_v[pl.ds(gg * _L, _L)]
            for el in range(_L):
                sp = task[el]
                rb = sp * _D
                db = (gg * _L + el) * _D
                for c in range(_D // _L):
                    rows_v[pl.ds(db + c * _L, _L)] = (
                        tab_v[pl.ds(rb + c * _L, _L)])
            return carry
        lax.fori_loop(0, _GPC, grp, jnp.int32(0))
        copy = pltpu.make_async_copy(
            rows_v.at[pl.ds(ch * _WCHUNK * _D, _WCHUNK * _D)],
            out_hbm.at[pl.ds((base + ch * _WCHUNK) * _D, _WCHUNK * _D)],
            sem)
        copy.start()
        copies.append(copy)
    for copy in copies:
        copy.wait()


def kernel(target_module, port_name, embedding_weight, lookup):
    tm = target_module.astype(jnp.int32)
    pn = port_name.astype(jnp.int32)
    lut = jnp.pad(lookup.astype(jnp.int32).reshape(-1), (0, 2))
    out = _task_encode_sc(tm, pn, embedding_weight.reshape(-1), lut)
    return out.reshape(_B, 1, _D)


# speedup vs baseline: 1.5424x; 1.5424x over previous
import functools

import jax
import jax.numpy as jnp
from jax import lax
from jax.experimental import pallas as pl
from jax.experimental.pallas import tpu as pltpu
from jax.experimental.pallas import tpu_sc as plsc

_B = 16384
_D = 128
_NT = 12
_NC = 2
_NS = 16
_NW = _NC * _NS
_BPW = _B // _NW
_L = 16
_G = _BPW // _L
_WCHUNK = 128
_NWCH = _BPW // _WCHUNK
_GPC = _WCHUNK // _L


@functools.partial(
    pl.kernel,
    out_type=jax.ShapeDtypeStruct((_B * _D,), jnp.float32),
    mesh=plsc.VectorSubcoreMesh(core_axis_name="c", subcore_axis_name="s"),
    compiler_params=pltpu.CompilerParams(needs_layout_passes=False),
    scratch_types=[
        pltpu.VMEM((_BPW,), jnp.int32),
        pltpu.VMEM((_BPW,), jnp.int32),
        pltpu.VMEM((_BPW,), jnp.int32),
        pltpu.VMEM((32,), jnp.int32),
        pltpu.VMEM((_NT * _D,), jnp.float32),
        pltpu.VMEM((_BPW * _D,), jnp.float32),
        pltpu.SemaphoreType.DMA,
    ],
)
def _task_encode_sc(tm_hbm, pn_hbm, table_hbm, lut_hbm, out_hbm,
                    tm_v, pn_v, idx_v, lut_v, tab_v, rows_v, sem):
    wid = lax.axis_index("s") * _NC + lax.axis_index("c")
    base = wid * _BPW

    pltpu.sync_copy(tm_hbm.at[pl.ds(base, _BPW)], tm_v)
    pltpu.sync_copy(pn_hbm.at[pl.ds(base, _BPW)], pn_v)
    pltpu.sync_copy(lut_hbm, lut_v)
    pltpu.sync_copy(table_hbm, tab_v)

    def lut_body(i, carry):
        tm = tm_v[pl.ds(i * _L, _L)]
        pn = pn_v[pl.ds(i * _L, _L)]
        idx_v[pl.ds(i * _L, _L)] = plsc.load_gather(lut_v, [tm * 3 + pn])
        return carry

    lax.fori_loop(0, _G, lut_body, jnp.int32(0), unroll=4)

    lanes = lax.iota(jnp.int32, _L)
    copies = []
    for ch in range(_NWCH):
        def grp(g, carry, ch=ch):
            gg = ch * _GPC + g
            task = idx_v[pl.ds(gg * _L, _L)]
            src0 = task * _D + lanes
            dst0 = gg * (_L * _D) + lanes * (_D + 1)
            for c in range(_D // _L):
                vals = plsc.load_gather(tab_v, [src0 + c * _L])
                plsc.store_scatter(rows_v, [dst0 + c * _L], vals)
            return carry
        lax.fori_loop(0, _GPC, grp, jnp.int32(0))
        pltpu.sync_copy(
            rows_v.at[pl.ds(ch * _WCHUNK * _D, _WCHUNK * _D)],
            out_hbm.at[pl.ds((base + ch * _WCHUNK) * _D, _WCHUNK * _D)])
    del copies


def kernel(target_module, port_name, embedding_weight, lookup):
    tm = target_module.astype(jnp.int32)
    pn = port_name.astype(jnp.int32)
    lut = jnp.pad(lookup.astype(jnp.int32).reshape(-1), (0, 2))
    out = _task_encode_sc(tm, pn, embedding_weight.reshape(-1), lut)
    return out.reshape(_B, 1, _D)


# speedup vs baseline: 1.6677x; 1.0812x over previous
import functools

import jax
import jax.numpy as jnp
from jax import lax
from jax.experimental import pallas as pl
from jax.experimental.pallas import tpu as pltpu
from jax.experimental.pallas import tpu_sc as plsc

_B = 16384
_D = 128
_NT = 12
_NC = 2
_NS = 16
_NW = _NC * _NS
_BPW = _B // _NW
_L = 16
_G = _BPW // _L
_WCHUNK = 128
_NWCH = _BPW // _WCHUNK
_GPC = _WCHUNK // _L


@functools.partial(
    pl.kernel,
    out_type=jax.ShapeDtypeStruct((_B * _D,), jnp.float32),
    mesh=plsc.VectorSubcoreMesh(core_axis_name="c", subcore_axis_name="s"),
    compiler_params=pltpu.CompilerParams(needs_layout_passes=False),
    scratch_types=[
        pltpu.VMEM((_BPW,), jnp.int32),
        pltpu.VMEM((_BPW,), jnp.int32),
        pltpu.VMEM((_BPW,), jnp.int32),
        pltpu.VMEM((32,), jnp.int32),
        pltpu.VMEM((_NT * _D,), jnp.float32),
        pltpu.VMEM((_BPW * _D,), jnp.float32),
        pltpu.SemaphoreType.DMA,
    ],
)
def _task_encode_sc(tm_hbm, pn_hbm, table_hbm, lut_hbm, out_hbm,
                    tm_v, pn_v, idx_v, lut_v, tab_v, rows_v, sem):
    wid = lax.axis_index("s") * _NC + lax.axis_index("c")
    base = wid * _BPW

    pltpu.sync_copy(tm_hbm.at[pl.ds(base, _BPW)], tm_v)
    pltpu.sync_copy(pn_hbm.at[pl.ds(base, _BPW)], pn_v)
    pltpu.sync_copy(lut_hbm, lut_v)
    pltpu.sync_copy(table_hbm, tab_v)

    def lut_body(i, carry):
        tm = tm_v[pl.ds(i * _L, _L)]
        pn = pn_v[pl.ds(i * _L, _L)]
        idx_v[pl.ds(i * _L, _L)] = plsc.load_gather(lut_v, [tm * 3 + pn])
        return carry

    lax.fori_loop(0, _G, lut_body, jnp.int32(0), unroll=4)

    lanes = lax.iota(jnp.int32, _L)

    def grp(gg, carry):
        task = idx_v[pl.ds(gg * _L, _L)]
        src0 = task * _D + lanes
        dst0 = gg * (_L * _D) + lanes * (_D + 1)
        for c in range(_D // _L):
            vals = plsc.load_gather(tab_v, [src0 + c * _L])
            plsc.store_scatter(rows_v, [dst0 + c * _L], vals)
        return carry

    lax.fori_loop(0, _G, grp, jnp.int32(0))

    copies = [
        pltpu.make_async_copy(
            rows_v.at[pl.ds(ch * _WCHUNK * _D, _WCHUNK * _D)],
            out_hbm.at[pl.ds((base + ch * _WCHUNK) * _D, _WCHUNK * _D)],
            sem)
        for ch in range(_NWCH)
    ]
    for copy in copies:
        copy.start()
    for copy in copies:
        copy.wait()


def kernel(target_module, port_name, embedding_weight, lookup):
    tm = target_module.astype(jnp.int32)
    pn = port_name.astype(jnp.int32)
    lut = jnp.pad(lookup.astype(jnp.int32).reshape(-1), (0, 2))
    out = _task_encode_sc(tm, pn, embedding_weight.reshape(-1), lut)
    return out.reshape(_B, 1, _D)
